# baseline (speedup 1.0000x reference)
import functools

import jax
import jax.numpy as jnp
from jax import lax
from jax.experimental import pallas as pl
from jax.experimental.pallas import tpu as pltpu
from jax.experimental.pallas import tpu_sc as plsc

B = 16384
D = 64
L = 16
NC, NS = 2, 16
NW = NC * NS
RPW = B // NW
CH = 128
NCH = RPW // CH
NBLK = RPW // L

_mesh = plsc.VectorSubcoreMesh(core_axis_name="c", subcore_axis_name="s")


@functools.partial(
    pl.kernel,
    mesh=_mesh,
    compiler_params=pltpu.CompilerParams(
        needs_layout_passes=False, use_tc_tiling_on_sc=False),
    out_type=jax.ShapeDtypeStruct((B,), jnp.float32),
    scratch_types=[
        pltpu.VMEM((NCH, CH), jnp.int32),
        pltpu.VMEM((NCH, CH), jnp.int32),
        pltpu.VMEM((RPW, D), jnp.float32),
        pltpu.VMEM((RPW, D), jnp.float32),
        pltpu.VMEM((2 * D + L,), jnp.float32),
        pltpu.VMEM((L * L,), jnp.float32),
        pltpu.VMEM((RPW,), jnp.float32),
        pltpu.SemaphoreType.DMA,
    ],
)
def _ncf_sc(uids, iids, utab, itab, wb, out,
            uidx_v, iidx_v, urows_v, irows_v, w_v, t_v, out_v, sem):
    wid = lax.axis_index("s") * NC + lax.axis_index("c")
    rb = wid * NCH
    pltpu.sync_copy(uids.at[pl.ds(rb, NCH)], uidx_v)
    pltpu.sync_copy(iids.at[pl.ds(rb, NCH)], iidx_v)
    pltpu.sync_copy(wb, w_v)
    handles = []
    for c in range(NCH):
        handles.append(pltpu.async_copy(
            utab.at[uidx_v.at[c]], urows_v.at[pl.ds(c * CH, CH)], sem))
        handles.append(pltpu.async_copy(
            itab.at[iidx_v.at[c]], irows_v.at[pl.ds(c * CH, CH)], sem))
    for h in handles:
        h.wait()

    wu = [w_v[pl.ds(16 * j, 16)] for j in range(D // 16)]
    wi = [w_v[pl.ds(D + 16 * j, 16)] for j in range(D // 16)]
    bias = w_v[pl.ds(2 * D, 16)]
    lanes16 = lax.iota(jnp.int32, L) * L

    def blk(b, carry):
        for r in range(L):
            row = b * L + r
            acc = urows_v[row, pl.ds(0, 16)] * wu[0]
            for j in range(1, D // 16):
                acc = acc + urows_v[row, pl.ds(16 * j, 16)] * wu[j]
            for j in range(D // 16):
                acc = acc + irows_v[row, pl.ds(16 * j, 16)] * wi[j]
            plsc.store_scatter(t_v, [lanes16 + r], acc)
        blockacc = bias
        for k in range(L):
            blockacc = blockacc + t_v[pl.ds(k * L, L)]
        out_v[pl.ds(b * L, L)] = blockacc
        return carry

    lax.fori_loop(0, NBLK, blk, 0)
    pltpu.sync_copy(out_v, out.at[pl.ds(wid * RPW, RPW)])


def kernel(user_ids, item_ids, user_table, item_table, fc_w, fc_b):
    wb = jnp.concatenate(
        [fc_w[:, 0], jnp.broadcast_to(fc_b, (L,))])
    uids2 = user_ids.reshape(NW * NCH, CH)
    iids2 = item_ids.reshape(NW * NCH, CH)
    out = _ncf_sc(uids2, iids2, user_table, item_table, wb)
    return out.reshape(B, 1)
